# Initial kernel scaffold; baseline (speedup 1.0000x reference)
import functools

import jax
import jax.numpy as jnp
from jax import lax
from jax.experimental import pallas as pl
from jax.experimental.pallas import tpu as pltpu
from jax.experimental.pallas import tpu_sc as plsc

N = 10000
E = 160000
D_IN = 256
HEADS = 8
HID = 8
OUT = 16

NPAD = 10240
BM = 512
NBLK = NPAD // BM
ETOT = E + N
NWORK = 32
CHUNK = 128
EPW = ((ETOT + NWORK * CHUNK - 1) // (NWORK * CHUNK)) * CHUNK
NCHUNK = EPW // CHUNK
EPAD = EPW * NWORK
ROWS_PER_TILE = NPAD // 16

_F32 = jnp.float32


def _prep1_body(x_ref, w1_ref, p1_ref, h_ref, sd_ref, m_ref):
    i = pl.program_id(0)
    h = jnp.dot(x_ref[...], w1_ref[...], preferred_element_type=_F32)
    h_ref[...] = h
    sd = jnp.dot(h, p1_ref[...], preferred_element_type=_F32)
    sd_ref[...] = sd
    bm = jnp.max(sd, axis=0, keepdims=True)

    @pl.when(i == 0)
    def _():
        m_ref[...] = bm

    @pl.when(i > 0)
    def _():
        m_ref[...] = jnp.maximum(m_ref[...], bm)

    @pl.when(i == NBLK - 1)
    def _():
        mm = m_ref[...]
        z = mm[:, :HEADS] + mm[:, HEADS:]
        zz = jnp.maximum(z, 0.2 * z)
        m_ref[...] = jnp.concatenate([zz, zz], axis=1)


def _tc_prep1(x_pad, w1, p1):
    return pl.pallas_call(
        _prep1_body,
        grid=(NBLK,),
        in_specs=[
            pl.BlockSpec((BM, D_IN), lambda i: (i, 0)),
            pl.BlockSpec((D_IN, HEADS * HID), lambda i: (0, 0)),
            pl.BlockSpec((HEADS * HID, 2 * HEADS), lambda i: (0, 0)),
        ],
        out_specs=[
            pl.BlockSpec((BM, HEADS * HID), lambda i: (i, 0)),
            pl.BlockSpec((BM, 2 * HEADS), lambda i: (i, 0)),
            pl.BlockSpec((1, 16), lambda i: (0, 0)),
        ],
        out_shape=[
            jax.ShapeDtypeStruct((NPAD, HEADS * HID), _F32),
            jax.ShapeDtypeStruct((NPAD, 2 * HEADS), _F32),
            jax.ShapeDtypeStruct((1, 16), _F32),
        ],
    )(x_pad, w1, p1)


def _mid_body(acc_ref, b1_ref, w2_ref, p2_ref, r8_ref, h2_ref, sd2_ref, m_ref):
    i = pl.program_id(0)
    a = acc_ref[0] + acc_ref[1]
    numer = a[:, : HEADS * HID]
    denom = a[:, HEADS * HID : HEADS * HID + HEADS]
    deb = jnp.dot(denom, r8_ref[...], preferred_element_type=_F32)
    o1 = numer / (deb + 1e-16) + b1_ref[...]
    a1 = jnp.where(o1 > 0, o1, jnp.exp(o1) - 1.0)
    h2 = jnp.dot(a1, w2_ref[...], preferred_element_type=_F32)
    h2_ref[...] = h2
    sd2 = jnp.dot(h2, p2_ref[...], preferred_element_type=_F32)
    sd2_ref[...] = sd2
    bm = jnp.max(sd2, axis=0, keepdims=True)

    @pl.when(i == 0)
    def _():
        m_ref[...] = bm

    @pl.when(i > 0)
    def _():
        m_ref[...] = jnp.maximum(m_ref[...], bm)

    @pl.when(i == NBLK - 1)
    def _():
        mm = m_ref[...]
        z = mm[0, 0] + mm[0, 1]
        zz = jnp.maximum(z, 0.2 * z)
        m_ref[...] = jnp.full((1, 16), zz, _F32)


def _tc_mid(acc1, b1, w2, p2, r8):
    return pl.pallas_call(
        _mid_body,
        grid=(NBLK,),
        in_specs=[
            pl.BlockSpec((2, BM, 80), lambda i: (0, i, 0)),
            pl.BlockSpec((1, HEADS * HID), lambda i: (0, 0)),
            pl.BlockSpec((HEADS * HID, OUT), lambda i: (0, 0)),
            pl.BlockSpec((OUT, 16), lambda i: (0, 0)),
            pl.BlockSpec((HEADS, HEADS * HID), lambda i: (0, 0)),
        ],
        out_specs=[
            pl.BlockSpec((BM, OUT), lambda i: (i, 0)),
            pl.BlockSpec((BM, 16), lambda i: (i, 0)),
            pl.BlockSpec((1, 16), lambda i: (0, 0)),
        ],
        out_shape=[
            jax.ShapeDtypeStruct((NPAD, OUT), _F32),
            jax.ShapeDtypeStruct((NPAD, 16), _F32),
            jax.ShapeDtypeStruct((1, 16), _F32),
        ],
    )(acc1, b1, w2, p2, r8)


def _fin_body(acc_ref, b2_ref, o_ref):
    a = acc_ref[0] + acc_ref[1]
    numer = a[:, :OUT]
    denom = a[:, OUT : OUT + 1]
    o = numer / (denom + 1e-16) + b2_ref[...]
    mx = jnp.max(o, axis=1, keepdims=True)
    ex = jnp.exp(o - mx)
    o_ref[...] = (o - mx) - jnp.log(jnp.sum(ex, axis=1, keepdims=True))


def _tc_fin(acc2, b2):
    return pl.pallas_call(
        _fin_body,
        grid=(NBLK,),
        in_specs=[
            pl.BlockSpec((2, BM, 32), lambda i: (0, i, 0)),
            pl.BlockSpec((1, OUT), lambda i: (0, 0)),
        ],
        out_specs=pl.BlockSpec((BM, OUT), lambda i: (i, 0)),
        out_shape=jax.ShapeDtypeStruct((NPAD, OUT), _F32),
    )(acc2, b2)


def _make_sc_edge(hw, msgw, heads8):
    mesh = plsc.VectorSubcoreMesh(core_axis_name="c", subcore_axis_name="s")
    wbuf_len = CHUNK * HEADS + 16 if heads8 else CHUNK + 16

    @functools.partial(
        pl.kernel,
        mesh=mesh,
        out_type=jax.ShapeDtypeStruct((2, NPAD, msgw), _F32),
        scratch_types=[
            pltpu.VMEM_SHARED((NPAD, msgw), _F32),
            pltpu.VMEM((EPW,), jnp.int32),
            pltpu.VMEM((EPW,), jnp.int32),
            pltpu.VMEM((CHUNK,), jnp.int32),
            pltpu.VMEM((CHUNK,), jnp.int32),
            pltpu.VMEM((CHUNK, 16), _F32),
            pltpu.VMEM((CHUNK, 16), _F32),
            pltpu.VMEM((CHUNK, hw), _F32),
            pltpu.VMEM((wbuf_len,), _F32),
            pltpu.VMEM((CHUNK, msgw), _F32),
            pltpu.VMEM((1, 16), _F32),
            pltpu.SemaphoreType.DMA,
            pltpu.SemaphoreType.DMA,
            pltpu.SemaphoreType.DMA,
        ],
    )
    def sc_edge(src_hbm, dst_hbm, sd_hbm, h_hbm, m_hbm, z_hbm, out_hbm,
                acc, src_all, dst_all, src_idx, dst_idx, sdv_s, sdv_d,
                hv, wbuf, msgbuf, mbuf, sem1, sem2, sem3):
        c = lax.axis_index("c")
        s = lax.axis_index("s")
        wid = c * 16 + s
        row0 = s * ROWS_PER_TILE
        pltpu.sync_copy(z_hbm.at[pl.ds(row0, ROWS_PER_TILE)],
                        acc.at[pl.ds(row0, ROWS_PER_TILE)])
        pltpu.sync_copy(m_hbm, mbuf)
        base = wid * EPW
        pltpu.sync_copy(src_hbm.at[pl.ds(base, EPW)], src_all)
        pltpu.sync_copy(dst_hbm.at[pl.ds(base, EPW)], dst_all)
        plsc.subcore_barrier()

        iota = lax.iota(jnp.int32, 16)
        zeros16 = iota * 0
        mvec = mbuf[0, :]
        rsel = lax.shift_right_logical(iota, 3)
        low3 = lax.bitwise_and(iota, 7)
        wbuf[pl.ds(wbuf_len - 16, 16)] = jnp.zeros((16,), _F32)

        @pl.loop(0, NCHUNK)
        def _chunk(k):
            off = k * CHUNK

            @pl.loop(0, CHUNK, step=16)
            def _cpidx(j):
                src_idx[pl.ds(j, 16)] = src_all[pl.ds(off + j, 16)]
                dst_idx[pl.ds(j, 16)] = dst_all[pl.ds(off + j, 16)]

            cp1 = pltpu.async_copy(sd_hbm.at[src_idx], sdv_s, sem1)
            cp2 = pltpu.async_copy(sd_hbm.at[dst_idx], sdv_d, sem2)
            cp3 = pltpu.async_copy(h_hbm.at[src_idx], hv, sem3)
            cp1.wait()
            cp2.wait()

            if heads8:
                @pl.loop(0, CHUNK, step=2)
                def _w2(e):
                    sv = plsc.load_gather(sdv_s, [e + rsel, low3])
                    dv = plsc.load_gather(sdv_d, [e + rsel, low3 + 8])
                    z = sv + dv
                    z = jnp.maximum(z, 0.2 * z) - mvec
                    wbuf[pl.ds(e * 8, 16)] = jnp.exp(z)
            else:
                @pl.loop(0, CHUNK, step=16)
                def _w16(e):
                    sv = plsc.load_gather(sdv_s, [e + iota, zeros16])
                    dv = plsc.load_gather(sdv_d, [e + iota, zeros16 + 1])
                    z = sv + dv
                    z = jnp.maximum(z, 0.2 * z) - mvec
                    wbuf[pl.ds(e, 16)] = jnp.exp(z)

            cp3.wait()

            if heads8:
                @pl.loop(0, CHUNK)
                def _msg8(e):
                    b8 = e * 8
                    for v in range(4):
                        wv = plsc.load_gather(wbuf, [b8 + rsel * 8 + 16 * v])
                        msgbuf[e, pl.ds(16 * v, 16)] = hv[e, pl.ds(16 * v, 16)] * wv
                    msgbuf[e, pl.ds(64, 16)] = wbuf[pl.ds(b8, 16)]
            else:
                @pl.loop(0, CHUNK)
                def _msg1(e):
                    ws = plsc.load_gather(wbuf, [e + zeros16])
                    msgbuf[e, pl.ds(0, 16)] = hv[e, :] * ws
                    msgbuf[e, pl.ds(16, 16)] = ws

            pltpu.sync_copy(msgbuf, acc.at[dst_idx], add=True)

        plsc.subcore_barrier()
        pltpu.sync_copy(acc.at[pl.ds(row0, ROWS_PER_TILE)],
                        out_hbm.at[c, pl.ds(row0, ROWS_PER_TILE)])

    return sc_edge


_sc_edge1 = _make_sc_edge(HEADS * HID, 80, True)
_sc_edge2 = _make_sc_edge(OUT, 32, False)


def kernel(x, edge_index, w1, att_src1, att_dst1, b1, w2, att_src2,
           att_dst2, b2):
    x_pad = jnp.pad(x, ((0, NPAD - N), (0, 0)))
    loops = jnp.arange(N, dtype=jnp.int32)
    pad_e = jnp.full((EPAD - ETOT,), N, jnp.int32)
    src = jnp.concatenate([edge_index[0].astype(jnp.int32), loops, pad_e])
    dst = jnp.concatenate([edge_index[1].astype(jnp.int32), loops, pad_e])

    eye8 = jnp.eye(HEADS, dtype=_F32)
    p1_src = (att_src1[:, :, None] * eye8[:, None, :]).reshape(HEADS * HID, HEADS)
    p1_dst = (att_dst1[:, :, None] * eye8[:, None, :]).reshape(HEADS * HID, HEADS)
    p1 = jnp.concatenate([p1_src, p1_dst], axis=1)
    p2 = jnp.concatenate(
        [att_src2.reshape(OUT, 1), att_dst2.reshape(OUT, 1),
         jnp.zeros((OUT, 14), _F32)], axis=1)
    r8 = (eye8[:, :, None] * jnp.ones((1, 1, HID), _F32)).reshape(
        HEADS, HEADS * HID)

    z80 = jnp.zeros((NPAD, 80), _F32)
    z32 = jnp.zeros((NPAD, 32), _F32)

    h1, sd1, m1 = _tc_prep1(x_pad, w1, p1)
    acc1 = _sc_edge1(src, dst, sd1, h1, m1, z80)
    h2, sd2, m2 = _tc_mid(acc1, b1.reshape(1, HEADS * HID), w2, p2, r8)
    acc2 = _sc_edge2(src, dst, sd2, h2, m2, z32)
    out = _tc_fin(acc2, b2.reshape(1, OUT))
    return out[:N]


# speedup vs baseline: 69.6439x; 69.6439x over previous
import dataclasses
import functools

import jax
import jax.numpy as jnp
from jax import lax
from jax.experimental import pallas as pl
from jax.experimental.pallas import tpu as pltpu
from jax.experimental.pallas import tpu_sc as plsc

N = 10000
E = 160000
D_IN = 256
HEADS = 8
HID = 8
OUT = 16

NPAD = 10240
BM = 512
NBLK = NPAD // BM
ETOT = E + N
NWORK = 32
CHUNK = 128
EPW = ((ETOT + NWORK * CHUNK - 1) // (NWORK * CHUNK)) * CHUNK
NCHUNK = EPW // CHUNK
EPAD = EPW * NWORK
ROWS_PER_TILE = NPAD // 16

_F32 = jnp.float32


def _prep1_body(x_ref, w1_ref, p1_ref, h_ref, sd_ref, m_ref):
    i = pl.program_id(0)
    h = jnp.dot(x_ref[...], w1_ref[...], preferred_element_type=_F32)
    h_ref[...] = h
    sd = jnp.dot(h, p1_ref[...], preferred_element_type=_F32)
    sd_ref[...] = sd
    bm = jnp.max(sd, axis=0, keepdims=True)

    @pl.when(i == 0)
    def _():
        m_ref[...] = bm

    @pl.when(i > 0)
    def _():
        m_ref[...] = jnp.maximum(m_ref[...], bm)

    @pl.when(i == NBLK - 1)
    def _():
        mm = m_ref[...]
        z = mm[:, :HEADS] + mm[:, HEADS:]
        zz = jnp.maximum(z, 0.2 * z)
        m_ref[...] = jnp.concatenate([zz, zz], axis=1)


def _tc_prep1(x_pad, w1, p1):
    return pl.pallas_call(
        _prep1_body,
        grid=(NBLK,),
        in_specs=[
            pl.BlockSpec((BM, D_IN), lambda i: (i, 0)),
            pl.BlockSpec((D_IN, HEADS * HID), lambda i: (0, 0)),
            pl.BlockSpec((HEADS * HID, 2 * HEADS), lambda i: (0, 0)),
        ],
        out_specs=[
            pl.BlockSpec((BM, HEADS * HID), lambda i: (i, 0)),
            pl.BlockSpec((BM, 2 * HEADS), lambda i: (i, 0)),
            pl.BlockSpec((1, 16), lambda i: (0, 0)),
        ],
        out_shape=[
            jax.ShapeDtypeStruct((NPAD, HEADS * HID), _F32),
            jax.ShapeDtypeStruct((NPAD, 2 * HEADS), _F32),
            jax.ShapeDtypeStruct((1, 16), _F32),
        ],
    )(x_pad, w1, p1)


def _mid_body(acc_ref, b1_ref, w2_ref, p2_ref, r8_ref, h2_ref, sd2_ref, m_ref):
    i = pl.program_id(0)
    a = acc_ref[0] + acc_ref[1]
    numer = a[:, : HEADS * HID]
    denom = a[:, HEADS * HID : HEADS * HID + HEADS]
    deb = jnp.dot(denom, r8_ref[...], preferred_element_type=_F32)
    o1 = numer / (deb + 1e-16) + b1_ref[...]
    a1 = jnp.where(o1 > 0, o1, jnp.exp(o1) - 1.0)
    h2 = jnp.dot(a1, w2_ref[...], preferred_element_type=_F32)
    h2_ref[...] = h2
    sd2 = jnp.dot(h2, p2_ref[...], preferred_element_type=_F32)
    sd2_ref[...] = sd2
    bm = jnp.max(sd2, axis=0, keepdims=True)

    @pl.when(i == 0)
    def _():
        m_ref[...] = bm

    @pl.when(i > 0)
    def _():
        m_ref[...] = jnp.maximum(m_ref[...], bm)

    @pl.when(i == NBLK - 1)
    def _():
        mm = m_ref[...]
        z = mm[0, 0] + mm[0, 1]
        zz = jnp.maximum(z, 0.2 * z)
        m_ref[...] = jnp.full((1, 16), zz, _F32)


def _tc_mid(acc1, b1, w2, p2, r8):
    return pl.pallas_call(
        _mid_body,
        grid=(NBLK,),
        in_specs=[
            pl.BlockSpec((2, BM, 80), lambda i: (0, i, 0)),
            pl.BlockSpec((1, HEADS * HID), lambda i: (0, 0)),
            pl.BlockSpec((HEADS * HID, OUT), lambda i: (0, 0)),
            pl.BlockSpec((OUT, 16), lambda i: (0, 0)),
            pl.BlockSpec((HEADS, HEADS * HID), lambda i: (0, 0)),
        ],
        out_specs=[
            pl.BlockSpec((BM, OUT), lambda i: (i, 0)),
            pl.BlockSpec((BM, 16), lambda i: (i, 0)),
            pl.BlockSpec((1, 16), lambda i: (0, 0)),
        ],
        out_shape=[
            jax.ShapeDtypeStruct((NPAD, OUT), _F32),
            jax.ShapeDtypeStruct((NPAD, 16), _F32),
            jax.ShapeDtypeStruct((1, 16), _F32),
        ],
    )(acc1, b1, w2, p2, r8)


def _fin_body(acc_ref, b2_ref, o_ref):
    a = acc_ref[0] + acc_ref[1]
    numer = a[:, :OUT]
    denom = a[:, OUT : OUT + 1]
    o = numer / (denom + 1e-16) + b2_ref[...]
    mx = jnp.max(o, axis=1, keepdims=True)
    ex = jnp.exp(o - mx)
    o_ref[...] = (o - mx) - jnp.log(jnp.sum(ex, axis=1, keepdims=True))


def _tc_fin(acc2, b2):
    return pl.pallas_call(
        _fin_body,
        grid=(NBLK,),
        in_specs=[
            pl.BlockSpec((2, BM, 32), lambda i: (0, i, 0)),
            pl.BlockSpec((1, OUT), lambda i: (0, 0)),
        ],
        out_specs=pl.BlockSpec((BM, OUT), lambda i: (i, 0)),
        out_shape=jax.ShapeDtypeStruct((NPAD, OUT), _F32),
    )(acc2, b2)


def _make_sc_edge_v2(hw, msgw, heads8):
    mesh = plsc.VectorSubcoreMesh(core_axis_name="c", subcore_axis_name="s")
    wbuf_len = CHUNK * HEADS + 16 if heads8 else CHUNK + 16
    cp = pltpu.CompilerParams(needs_layout_passes=False,
                              use_tc_tiling_on_sc=False)

    @functools.partial(
        pl.kernel,
        mesh=mesh,
        compiler_params=cp,
        out_type=jax.ShapeDtypeStruct((2, NPAD, msgw), _F32),
        scratch_types=[
            pltpu.VMEM_SHARED((NPAD, msgw), _F32),
            pltpu.VMEM((EPW,), jnp.int32),
            pltpu.VMEM((EPW,), jnp.int32),
            [pltpu.VMEM((CHUNK,), jnp.int32)] * 2,
            [pltpu.VMEM((CHUNK,), jnp.int32)] * 2,
            [pltpu.VMEM((CHUNK, 16), _F32)] * 2,
            [pltpu.VMEM((CHUNK, 16), _F32)] * 2,
            [pltpu.VMEM((CHUNK, hw), _F32)] * 2,
            pltpu.VMEM((CHUNK, msgw), _F32),
            pltpu.VMEM((wbuf_len,), _F32),
            pltpu.VMEM((1, 16), _F32),
            [pltpu.SemaphoreType.DMA] * 2,
        ],
    )
    def sc_edge(src_hbm, dst_hbm, sd_hbm, h_hbm, m_hbm, z_hbm, out_hbm,
                acc, src_all, dst_all, src_idx, dst_idx,
                sdv_s, sdv_d, hv, msg, wbuf, mbuf, gsem):
        c = lax.axis_index("c")
        s = lax.axis_index("s")
        wid = c * 16 + s
        row0 = s * ROWS_PER_TILE
        pltpu.sync_copy(z_hbm.at[pl.ds(row0, ROWS_PER_TILE)],
                        acc.at[pl.ds(row0, ROWS_PER_TILE)])
        pltpu.sync_copy(m_hbm, mbuf)
        base = wid * EPW
        pltpu.sync_copy(src_hbm.at[pl.ds(base, EPW)], src_all)
        pltpu.sync_copy(dst_hbm.at[pl.ds(base, EPW)], dst_all)
        plsc.subcore_barrier()

        iota = lax.iota(jnp.int32, 16)
        zeros16 = iota * 0
        mvec = mbuf[0, :]
        rsel = lax.shift_right_logical(iota, 3)
        low3 = lax.bitwise_and(iota, 7)
        wbuf[pl.ds(wbuf_len - 16, 16)] = jnp.zeros((16,), _F32)

        def prefetch(k, b):
            off = k * CHUNK

            @pl.loop(0, CHUNK, step=16)
            def _cp(j):
                src_idx[b][pl.ds(j, 16)] = src_all[pl.ds(off + j, 16)]
                dst_idx[b][pl.ds(j, 16)] = dst_all[pl.ds(off + j, 16)]

            pltpu.async_copy(sd_hbm.at[src_idx[b]], sdv_s[b], gsem[b])
            pltpu.async_copy(sd_hbm.at[dst_idx[b]], sdv_d[b], gsem[b])
            pltpu.async_copy(h_hbm.at[src_idx[b]], hv[b], gsem[b])

        def half(k, b):
            pltpu.make_async_copy(sd_hbm.at[src_idx[b]], sdv_s[b], gsem[b]).wait()
            pltpu.make_async_copy(sd_hbm.at[dst_idx[b]], sdv_d[b], gsem[b]).wait()
            pltpu.make_async_copy(h_hbm.at[src_idx[b]], hv[b], gsem[b]).wait()

            if heads8:
                @pl.loop(0, CHUNK, step=2)
                def _w2(e):
                    sv = plsc.load_gather(sdv_s[b], [e + rsel, low3])
                    dv = plsc.load_gather(sdv_d[b], [e + rsel, low3 + 8])
                    z = sv + dv
                    z = jnp.maximum(z, 0.2 * z) - mvec
                    wbuf[pl.ds(e * 8, 16)] = jnp.exp(z)

                @pl.loop(0, CHUNK)
                def _msg8(e):
                    wp = plsc.load_gather(wbuf, [e * 8 + low3])
                    for v in range(4):
                        msg[e, pl.ds(16 * v, 16)] = hv[b][e, pl.ds(16 * v, 16)] * wp
                    msg[e, pl.ds(64, 16)] = wp
            else:
                @pl.loop(0, CHUNK, step=16)
                def _w16(e):
                    sv = plsc.load_gather(sdv_s[b], [e + iota, zeros16])
                    dv = plsc.load_gather(sdv_d[b], [e + iota, zeros16 + 1])
                    z = sv + dv
                    z = jnp.maximum(z, 0.2 * z) - mvec
                    wbuf[pl.ds(e, 16)] = jnp.exp(z)

                @pl.loop(0, CHUNK)
                def _msg1(e):
                    ws = plsc.load_gather(wbuf, [e + zeros16])
                    msg[e, pl.ds(0, 16)] = hv[b][e, :] * ws
                    msg[e, pl.ds(16, 16)] = ws

            pltpu.sync_copy(msg, acc.at[dst_idx[b]], add=True)

            @pl.when(k + 2 < NCHUNK)
            def _():
                prefetch(k + 2, b)

        prefetch(0, 0)
        prefetch(1, 1)

        @pl.loop(0, NCHUNK, step=2)
        def _loop(k):
            half(k, 0)
            half(k + 1, 1)

        plsc.subcore_barrier()
        pltpu.sync_copy(acc.at[pl.ds(row0, ROWS_PER_TILE)],
                        out_hbm.at[c, pl.ds(row0, ROWS_PER_TILE)])

    return sc_edge


_sc_edge1 = _make_sc_edge_v2(HEADS * HID, 80, True)
_sc_edge2 = _make_sc_edge_v2(OUT, 32, False)


def kernel(x, edge_index, w1, att_src1, att_dst1, b1, w2, att_src2,
           att_dst2, b2):
    x_pad = jnp.pad(x, ((0, NPAD - N), (0, 0)))
    loops = jnp.arange(N, dtype=jnp.int32)
    pad_e = jnp.full((EPAD - ETOT,), N, jnp.int32)
    src = jnp.concatenate([edge_index[0].astype(jnp.int32), loops, pad_e])
    dst = jnp.concatenate([edge_index[1].astype(jnp.int32), loops, pad_e])

    eye8 = jnp.eye(HEADS, dtype=_F32)
    w1p = w1.reshape(D_IN, HEADS, HID).transpose(0, 2, 1).reshape(
        D_IN, HEADS * HID)
    p1_src = (att_src1.T[:, :, None] * eye8[None, :, :]).reshape(
        HEADS * HID, HEADS)
    p1_dst = (att_dst1.T[:, :, None] * eye8[None, :, :]).reshape(
        HEADS * HID, HEADS)
    p1 = jnp.concatenate([p1_src, p1_dst], axis=1)
    p2 = jnp.concatenate(
        [att_src2.reshape(OUT, 1), att_dst2.reshape(OUT, 1),
         jnp.zeros((OUT, 14), _F32)], axis=1)
    r8 = jnp.tile(eye8, (1, HID))
    b1p = b1.reshape(HEADS, HID).T.reshape(1, HEADS * HID)
    w2p = w2.reshape(HEADS, HID, OUT).transpose(1, 0, 2).reshape(
        HEADS * HID, OUT)

    z80 = jnp.zeros((NPAD, 80), _F32)
    z32 = jnp.zeros((NPAD, 32), _F32)

    h1, sd1, m1 = _tc_prep1(x_pad, w1p, p1)
    acc1 = _sc_edge1(src, dst, sd1, h1, m1, z80)
    h2, sd2, m2 = _tc_mid(acc1, b1p, w2p, p2, r8)
    acc2 = _sc_edge2(src, dst, sd2, h2, m2, z32)
    out = _tc_fin(acc2, b2.reshape(1, OUT))
    return out[:N]


# speedup vs baseline: 92.4538x; 1.3275x over previous
import dataclasses
import functools

import jax
import jax.numpy as jnp
from jax import lax
from jax.experimental import pallas as pl
from jax.experimental.pallas import tpu as pltpu
from jax.experimental.pallas import tpu_sc as plsc

N = 10000
E = 160000
D_IN = 256
HEADS = 8
HID = 8
OUT = 16

NPAD = 10240
BM = 512
NBLK = NPAD // BM
ETOT = E + N
NWORK = 32
CHUNK = 128
EPW = ((ETOT + NWORK * CHUNK - 1) // (NWORK * CHUNK)) * CHUNK
NCHUNK = EPW // CHUNK
EPAD = EPW * NWORK
ROWS_PER_TILE = NPAD // 16

_F32 = jnp.float32


def _prep1_body(x_ref, w1_ref, p1_ref, h_ref, sd_ref, m_ref):
    i = pl.program_id(0)
    h = jnp.dot(x_ref[...], w1_ref[...], preferred_element_type=_F32)
    h_ref[...] = h
    sd = jnp.dot(h, p1_ref[...], preferred_element_type=_F32)
    sd_ref[...] = sd
    bm = jnp.max(sd, axis=0, keepdims=True)

    @pl.when(i == 0)
    def _():
        m_ref[...] = bm

    @pl.when(i > 0)
    def _():
        m_ref[...] = jnp.maximum(m_ref[...], bm)

    @pl.when(i == NBLK - 1)
    def _():
        mm = m_ref[...]
        z = mm[:, :HEADS] + mm[:, HEADS:]
        zz = jnp.maximum(z, 0.2 * z)
        m_ref[...] = jnp.concatenate([zz, zz], axis=1)


def _tc_prep1(x_pad, w1, p1):
    return pl.pallas_call(
        _prep1_body,
        grid=(NBLK,),
        in_specs=[
            pl.BlockSpec((BM, D_IN), lambda i: (i, 0)),
            pl.BlockSpec((D_IN, HEADS * HID), lambda i: (0, 0)),
            pl.BlockSpec((HEADS * HID, 2 * HEADS), lambda i: (0, 0)),
        ],
        out_specs=[
            pl.BlockSpec((BM, HEADS * HID), lambda i: (i, 0)),
            pl.BlockSpec((BM, 2 * HEADS), lambda i: (i, 0)),
            pl.BlockSpec((1, 16), lambda i: (0, 0)),
        ],
        out_shape=[
            jax.ShapeDtypeStruct((NPAD, HEADS * HID), _F32),
            jax.ShapeDtypeStruct((NPAD, 2 * HEADS), _F32),
            jax.ShapeDtypeStruct((1, 16), _F32),
        ],
    )(x_pad, w1, p1)


def _mid_body(acc_ref, b1_ref, w2_ref, p2_ref, r8_ref, h2_ref, sd2_ref, m_ref):
    i = pl.program_id(0)
    a = acc_ref[0] + acc_ref[1]
    numer = a[:, : HEADS * HID]
    denom = a[:, HEADS * HID : HEADS * HID + HEADS]
    deb = jnp.dot(denom, r8_ref[...], preferred_element_type=_F32)
    o1 = numer / (deb + 1e-16) + b1_ref[...]
    a1 = jnp.where(o1 > 0, o1, jnp.exp(o1) - 1.0)
    h2 = jnp.dot(a1, w2_ref[...], preferred_element_type=_F32)
    h2_ref[...] = h2
    sd2 = jnp.dot(h2, p2_ref[...], preferred_element_type=_F32)
    sd2_ref[...] = sd2
    bm = jnp.max(sd2, axis=0, keepdims=True)

    @pl.when(i == 0)
    def _():
        m_ref[...] = bm

    @pl.when(i > 0)
    def _():
        m_ref[...] = jnp.maximum(m_ref[...], bm)

    @pl.when(i == NBLK - 1)
    def _():
        mm = m_ref[...]
        z = mm[0, 0] + mm[0, 1]
        zz = jnp.maximum(z, 0.2 * z)
        m_ref[...] = jnp.full((1, 16), zz, _F32)


def _tc_mid(acc1, b1, w2, p2, r8):
    return pl.pallas_call(
        _mid_body,
        grid=(NBLK,),
        in_specs=[
            pl.BlockSpec((2, BM, 80), lambda i: (0, i, 0)),
            pl.BlockSpec((1, HEADS * HID), lambda i: (0, 0)),
            pl.BlockSpec((HEADS * HID, OUT), lambda i: (0, 0)),
            pl.BlockSpec((OUT, 16), lambda i: (0, 0)),
            pl.BlockSpec((HEADS, HEADS * HID), lambda i: (0, 0)),
        ],
        out_specs=[
            pl.BlockSpec((BM, OUT), lambda i: (i, 0)),
            pl.BlockSpec((BM, 16), lambda i: (i, 0)),
            pl.BlockSpec((1, 16), lambda i: (0, 0)),
        ],
        out_shape=[
            jax.ShapeDtypeStruct((NPAD, OUT), _F32),
            jax.ShapeDtypeStruct((NPAD, 16), _F32),
            jax.ShapeDtypeStruct((1, 16), _F32),
        ],
    )(acc1, b1, w2, p2, r8)


def _fin_body(acc_ref, b2_ref, o_ref):
    a = acc_ref[0] + acc_ref[1]
    numer = a[:, :OUT]
    denom = a[:, OUT : OUT + 1]
    o = numer / (denom + 1e-16) + b2_ref[...]
    mx = jnp.max(o, axis=1, keepdims=True)
    ex = jnp.exp(o - mx)
    o_ref[...] = (o - mx) - jnp.log(jnp.sum(ex, axis=1, keepdims=True))


def _tc_fin(acc2, b2):
    return pl.pallas_call(
        _fin_body,
        grid=(NBLK,),
        in_specs=[
            pl.BlockSpec((2, BM, 32), lambda i: (0, i, 0)),
            pl.BlockSpec((1, OUT), lambda i: (0, 0)),
        ],
        out_specs=pl.BlockSpec((BM, OUT), lambda i: (i, 0)),
        out_shape=jax.ShapeDtypeStruct((NPAD, OUT), _F32),
    )(acc2, b2)


def _make_sc_edge_v2(hw, msgw, heads8):
    mesh = plsc.VectorSubcoreMesh(core_axis_name="c", subcore_axis_name="s")
    wbuf_len = CHUNK * HEADS + 16 if heads8 else CHUNK + 16
    cp = pltpu.CompilerParams(needs_layout_passes=False,
                              use_tc_tiling_on_sc=False)

    @functools.partial(
        pl.kernel,
        mesh=mesh,
        compiler_params=cp,
        out_type=jax.ShapeDtypeStruct((2, NPAD, msgw), _F32),
        scratch_types=[
            pltpu.VMEM_SHARED((NPAD, msgw), _F32),
            pltpu.VMEM((EPW,), jnp.int32),
            pltpu.VMEM((EPW,), jnp.int32),
            [pltpu.VMEM((CHUNK,), jnp.int32)] * 2,
            [pltpu.VMEM((CHUNK,), jnp.int32)] * 2,
            [pltpu.VMEM((CHUNK, 16), _F32)] * 2,
            [pltpu.VMEM((CHUNK, 16), _F32)] * 2,
            [pltpu.VMEM((CHUNK, hw), _F32)] * 2,
            [pltpu.VMEM((CHUNK, msgw), _F32)] * 2,
            [pltpu.VMEM((CHUNK,), jnp.int32)] * 2,
            pltpu.VMEM((wbuf_len,), _F32),
            pltpu.VMEM((1, 16), _F32),
            [pltpu.SemaphoreType.DMA] * 2,
            [pltpu.SemaphoreType.DMA] * 2,
        ],
    )
    def sc_edge(src_hbm, dst_hbm, sd_hbm, h_hbm, m_hbm, z_hbm, out_hbm,
                acc, src_all, dst_all, src_idx, dst_idx,
                sdv_s, sdv_d, hv, msg, dsc_idx, wbuf, mbuf, gsem, ssem):
        c = lax.axis_index("c")
        s = lax.axis_index("s")
        wid = c * 16 + s
        row0 = s * ROWS_PER_TILE
        pltpu.sync_copy(z_hbm.at[pl.ds(row0, ROWS_PER_TILE)],
                        acc.at[pl.ds(row0, ROWS_PER_TILE)])
        pltpu.sync_copy(m_hbm, mbuf)
        base = wid * EPW
        pltpu.sync_copy(src_hbm.at[pl.ds(base, EPW)], src_all)
        pltpu.sync_copy(dst_hbm.at[pl.ds(base, EPW)], dst_all)
        plsc.subcore_barrier()

        iota = lax.iota(jnp.int32, 16)
        zeros16 = iota * 0
        mvec = mbuf[0, :]
        rsel = lax.shift_right_logical(iota, 3)
        low3 = lax.bitwise_and(iota, 7)
        wbuf[pl.ds(wbuf_len - 16, 16)] = jnp.zeros((16,), _F32)

        def prefetch(k, b):
            off = k * CHUNK

            @pl.loop(0, CHUNK, step=16)
            def _cp(j):
                src_idx[b][pl.ds(j, 16)] = src_all[pl.ds(off + j, 16)]
                dst_idx[b][pl.ds(j, 16)] = dst_all[pl.ds(off + j, 16)]

            pltpu.async_copy(sd_hbm.at[src_idx[b]], sdv_s[b], gsem[b])
            pltpu.async_copy(sd_hbm.at[dst_idx[b]], sdv_d[b], gsem[b])
            pltpu.async_copy(h_hbm.at[src_idx[b]], hv[b], gsem[b])

        def half(k, b):
            pltpu.make_async_copy(sd_hbm.at[src_idx[b]], sdv_s[b], gsem[b]).wait()
            pltpu.make_async_copy(sd_hbm.at[dst_idx[b]], sdv_d[b], gsem[b]).wait()
            pltpu.make_async_copy(h_hbm.at[src_idx[b]], hv[b], gsem[b]).wait()

            @pl.when(k >= 2)
            def _():
                pltpu.make_async_copy(msg[b], acc.at[dsc_idx[b]], ssem[b]).wait()

            if heads8:
                @plsc.parallel_loop(0, CHUNK, step=2, unroll=2)
                def _w2(e):
                    sv = plsc.load_gather(sdv_s[b], [e + rsel, low3])
                    dv = plsc.load_gather(sdv_d[b], [e + rsel, low3 + 8])
                    z = sv + dv
                    z = jnp.maximum(z, 0.2 * z) - mvec
                    wbuf[pl.ds(e * 8, 16)] = jnp.exp(z)

                @plsc.parallel_loop(0, CHUNK, step=1, unroll=2)
                def _msg8(e):
                    wp = plsc.load_gather(wbuf, [e * 8 + low3])
                    for v in range(4):
                        msg[b][e, pl.ds(16 * v, 16)] = hv[b][e, pl.ds(16 * v, 16)] * wp
                    msg[b][e, pl.ds(64, 16)] = wp
            else:
                @plsc.parallel_loop(0, CHUNK, step=16, unroll=2)
                def _w16(e):
                    sv = plsc.load_gather(sdv_s[b], [e + iota, zeros16])
                    dv = plsc.load_gather(sdv_d[b], [e + iota, zeros16 + 1])
                    z = sv + dv
                    z = jnp.maximum(z, 0.2 * z) - mvec
                    wbuf[pl.ds(e, 16)] = jnp.exp(z)

                @plsc.parallel_loop(0, CHUNK, step=1, unroll=2)
                def _msg1(e):
                    ws = plsc.load_gather(wbuf, [e + zeros16])
                    msg[b][e, pl.ds(0, 16)] = hv[b][e, :] * ws
                    msg[b][e, pl.ds(16, 16)] = ws

            @pl.loop(0, CHUNK, step=16)
            def _cpsc(j):
                dsc_idx[b][pl.ds(j, 16)] = dst_idx[b][pl.ds(j, 16)]

            pltpu.async_copy(msg[b], acc.at[dsc_idx[b]], ssem[b], add=True)

            @pl.when(k + 2 < NCHUNK)
            def _():
                prefetch(k + 2, b)

        prefetch(0, 0)
        prefetch(1, 1)

        @pl.loop(0, NCHUNK, step=2)
        def _loop(k):
            half(k, 0)
            half(k + 1, 1)

        pltpu.make_async_copy(msg[0], acc.at[dsc_idx[0]], ssem[0]).wait()
        pltpu.make_async_copy(msg[1], acc.at[dsc_idx[1]], ssem[1]).wait()
        plsc.subcore_barrier()
        pltpu.sync_copy(acc.at[pl.ds(row0, ROWS_PER_TILE)],
                        out_hbm.at[c, pl.ds(row0, ROWS_PER_TILE)])

    return sc_edge


_sc_edge1 = _make_sc_edge_v2(HEADS * HID, 80, True)
_sc_edge2 = _make_sc_edge_v2(OUT, 32, False)


def kernel(x, edge_index, w1, att_src1, att_dst1, b1, w2, att_src2,
           att_dst2, b2):
    x_pad = jnp.pad(x, ((0, NPAD - N), (0, 0)))
    loops = jnp.arange(N, dtype=jnp.int32)
    pad_e = jnp.full((EPAD - ETOT,), N, jnp.int32)
    src = jnp.concatenate([edge_index[0].astype(jnp.int32), loops, pad_e])
    dst = jnp.concatenate([edge_index[1].astype(jnp.int32), loops, pad_e])

    eye8 = jnp.eye(HEADS, dtype=_F32)
    w1p = w1.reshape(D_IN, HEADS, HID).transpose(0, 2, 1).reshape(
        D_IN, HEADS * HID)
    p1_src = (att_src1.T[:, :, None] * eye8[None, :, :]).reshape(
        HEADS * HID, HEADS)
    p1_dst = (att_dst1.T[:, :, None] * eye8[None, :, :]).reshape(
        HEADS * HID, HEADS)
    p1 = jnp.concatenate([p1_src, p1_dst], axis=1)
    p2 = jnp.concatenate(
        [att_src2.reshape(OUT, 1), att_dst2.reshape(OUT, 1),
         jnp.zeros((OUT, 14), _F32)], axis=1)
    r8 = jnp.tile(eye8, (1, HID))
    b1p = b1.reshape(HEADS, HID).T.reshape(1, HEADS * HID)
    w2p = w2.reshape(HEADS, HID, OUT).transpose(1, 0, 2).reshape(
        HEADS * HID, OUT)

    z80 = jnp.zeros((NPAD, 80), _F32)
    z32 = jnp.zeros((NPAD, 32), _F32)

    h1, sd1, m1 = _tc_prep1(x_pad, w1p, p1)
    acc1 = _sc_edge1(src, dst, sd1, h1, m1, z80)
    h2, sd2, m2 = _tc_mid(acc1, b1p, w2p, p2, r8)
    acc2 = _sc_edge2(src, dst, sd2, h2, m2, z32)
    out = _tc_fin(acc2, b2.reshape(1, OUT))
    return out[:N]
